# baseline (speedup 1.0000x reference)
import functools

import jax
import jax.numpy as jnp
from jax import lax
from jax.experimental import pallas as pl
from jax.experimental.pallas import tpu as pltpu
from jax.experimental.pallas import tpu_sc as plsc

B = 100
NPER = 100
N = B * NPER
E = 160000
D = 128
LAT = 56
T = 4
ATOM_V = 100
PIECE_V = 500
POS_V = 100
NPROP = 3

NC = 2
NS = 16
NW = NC * NS
CH = 128
SPM_ROWS = 10240
PN = N
ZROWS = 32
NBUF = 2
OUT_PER_SUB = 640

_EPE = 163840
_KE = _EPE // (NW * CH)
HREL_ROWS = 4 * N + 8


@functools.lru_cache(maxsize=None)
def _make_sc_segsum(k, dcols=D, ch=CH):
    mesh = plsc.VectorSubcoreMesh(core_axis_name="c", subcore_axis_name="s",
                                  num_cores=NC, num_subcores=NS)

    @functools.partial(
        pl.kernel,
        out_type=jax.ShapeDtypeStruct((NC, PN, D), jnp.float32),
        mesh=mesh,
        scratch_types=[
            pltpu.VMEM((k, ch), jnp.int32),
            pltpu.VMEM((k, ch), jnp.int32),
            pltpu.VMEM((NBUF * ch, dcols), jnp.float32),
            pltpu.VMEM((ZROWS, D), jnp.float32),
            pltpu.VMEM_SHARED((SPM_ROWS, D), jnp.float32),
            pltpu.SemaphoreType.DMA,
            pltpu.SemaphoreType.DMA,
            pltpu.SemaphoreType.DMA,
            pltpu.SemaphoreType.DMA,
        ],
    )
    def sc_segsum(gidx_hbm, dst_hbm, table_hbm, out_hbm,
                  gidx_v, dst_v, ring, zbuf, acc, s0, s1, s2, s3):
        c = lax.axis_index("c")
        s = lax.axis_index("s")
        wid = c * NS + s
        sems = (s0, s1)
        ssems = (s2, s3)

        pltpu.sync_copy(gidx_hbm.at[wid], gidx_v)
        pltpu.sync_copy(dst_hbm.at[wid], dst_v)

        def gather(j, b):
            return pltpu.make_async_copy(
                table_hbm.at[gidx_v.at[j]],
                ring.at[pl.ds(b * ch, ch)], sems[b])

        for b in range(NBUF):
            gather(b, b).start()

        zv = jnp.zeros((16,), jnp.float32)

        @pl.loop(0, ZROWS)
        def _(r):
            for u in range(D // 16):
                zbuf[r, pl.ds(u * 16, 16)] = zv

        zper = SPM_ROWS // NS

        @pl.loop(0, zper // ZROWS)
        def _(j):
            pltpu.sync_copy(zbuf, acc.at[pl.ds(s * zper + j * ZROWS, ZROWS)])

        plsc.subcore_barrier()

        @pl.loop(0, k, step=NBUF)
        def _(j):
            for b in range(NBUF):
                gather(j + b, b).wait()
                pltpu.sync_copy(ring.at[pl.ds(b * ch, ch)],
                                acc.at[dst_v.at[j + b]], add=True)

                @pl.when(j + b + NBUF < k)
                def _():
                    gather(j + b + NBUF, b).start()

        plsc.subcore_barrier()

        @pl.when(s < NS - 1)
        def _():
            pltpu.sync_copy(
                acc.at[pl.ds(s * OUT_PER_SUB, OUT_PER_SUB)],
                out_hbm.at[c].at[pl.ds(s * OUT_PER_SUB, OUT_PER_SUB)])

        @pl.when(s == NS - 1)
        def _():
            pltpu.sync_copy(
                acc.at[pl.ds((NS - 1) * OUT_PER_SUB, N - (NS - 1) * OUT_PER_SUB)],
                out_hbm.at[c].at[pl.ds((NS - 1) * OUT_PER_SUB,
                                       N - (NS - 1) * OUT_PER_SUB)])

    return sc_segsum


_RB = 400
_NB = N // _RB


def _emb_body(xi_r, xp_r, xo_r, ae_r, pe_r, oe_r, ee_r, h_r, hr_r, hs):
    @pl.when(pl.program_id(1) == 0)
    def _():
        def oh_dot(idx_r, tbl_r, v):
            oh = (idx_r[...][:, 0][:, None] ==
                  lax.broadcasted_iota(jnp.int32, (_RB, v), 1)
                  ).astype(jnp.float32)
            return jnp.dot(oh, tbl_r[...], preferred_element_type=jnp.float32)

        h0 = (oh_dot(xi_r, ae_r, ATOM_V) + oh_dot(xp_r, pe_r, PIECE_V)
              + oh_dot(xo_r, oe_r, POS_V))
        hs[...] = h0
        h_r[...] = h0

    hr_r[...] = jnp.maximum(hs[...] + ee_r[...].reshape(1, D), 0.0)


def _emb_build(xi, xp, xo, atom_emb, piece_emb, pos_emb, eemb):
    row1 = lambda i, a: (i, 0)
    fix = lambda i, a: (0, 0)
    return pl.pallas_call(
        _emb_body,
        grid=(_NB, 4),
        in_specs=[
            pl.BlockSpec((_RB, 1), row1),
            pl.BlockSpec((_RB, 1), row1),
            pl.BlockSpec((_RB, 1), row1),
            pl.BlockSpec((ATOM_V, D), fix),
            pl.BlockSpec((PIECE_V, D), fix),
            pl.BlockSpec((POS_V, D), fix),
            pl.BlockSpec((1, 1, D), lambda i, a: (a, 0, 0)),
        ],
        out_specs=[
            pl.BlockSpec((_RB, D), row1),
            pl.BlockSpec((_RB, D), lambda i, a: (a * _NB + i, 0)),
        ],
        out_shape=[
            jax.ShapeDtypeStruct((N, D), jnp.float32),
            jax.ShapeDtypeStruct((HREL_ROWS, D), jnp.float32),
        ],
        scratch_shapes=[pltpu.VMEM((_RB, D), jnp.float32)],
    )(xi, xp, xo, atom_emb, piece_emb, pos_emb, eemb)


def _mlp_math(h_r, parts_r, w1_r, b1_r, w2_r, b2_r, e_r):
    pre = (1.0 + e_r[0, 0]) * h_r[...] + parts_r[0] + parts_r[1]
    act = jnp.maximum(
        jnp.dot(pre, w1_r[...], preferred_element_type=jnp.float32) + b1_r[...],
        0.0)
    return jnp.dot(act, w2_r[...], preferred_element_type=jnp.float32) + b2_r[...]


def _mlp_table_body(h_r, parts_r, w1_r, b1_r, w2_r, b2_r, e_r, ee_r,
                    hn_r, hr_r, hs):
    @pl.when(pl.program_id(1) == 0)
    def _():
        hn = _mlp_math(h_r, parts_r, w1_r, b1_r, w2_r, b2_r, e_r)
        hs[...] = hn
        hn_r[...] = hn

    hr_r[...] = jnp.maximum(hs[...] + ee_r[...].reshape(1, D), 0.0)


def _mlp_table(h, parts, w1, b1, w2, b2, epsv, eemb):
    row = lambda i, a: (i, 0)
    fix = lambda i, a: (0, 0)
    return pl.pallas_call(
        _mlp_table_body,
        grid=(_NB, 4),
        in_specs=[
            pl.BlockSpec((_RB, D), row),
            pl.BlockSpec((NC, _RB, D), lambda i, a: (0, i, 0)),
            pl.BlockSpec((D, D), fix),
            pl.BlockSpec((1, D), fix),
            pl.BlockSpec((D, D), fix),
            pl.BlockSpec((1, D), fix),
            pl.BlockSpec(memory_space=pltpu.SMEM),
            pl.BlockSpec((1, 1, D), lambda i, a: (a, 0, 0)),
        ],
        out_specs=[
            pl.BlockSpec((_RB, D), row),
            pl.BlockSpec((_RB, D), lambda i, a: (a * _NB + i, 0)),
        ],
        out_shape=[
            jax.ShapeDtypeStruct((N, D), jnp.float32),
            jax.ShapeDtypeStruct((HREL_ROWS, D), jnp.float32),
        ],
        scratch_shapes=[pltpu.VMEM((_RB, D), jnp.float32)],
    )(h, parts, w1, b1, w2, b2, epsv, eemb)


def _mlp_final_body(h_r, parts_r, w1_r, b1_r, w2_r, b2_r, e_r, hn_r):
    hn_r[...] = _mlp_math(h_r, parts_r, w1_r, b1_r, w2_r, b2_r, e_r)


def _mlp_final(h, parts, w1, b1, w2, b2, epsv):
    row = lambda i: (i, 0)
    fix = lambda i: (0, 0)
    return pl.pallas_call(
        _mlp_final_body,
        grid=(_NB,),
        in_specs=[
            pl.BlockSpec((_RB, D), row),
            pl.BlockSpec((NC, _RB, D), lambda i: (0, i, 0)),
            pl.BlockSpec((D, D), fix),
            pl.BlockSpec((1, D), fix),
            pl.BlockSpec((D, D), fix),
            pl.BlockSpec((1, D), fix),
            pl.BlockSpec(memory_space=pltpu.SMEM),
        ],
        out_specs=pl.BlockSpec((_RB, D), row),
        out_shape=jax.ShapeDtypeStruct((N, D), jnp.float32),
    )(h, parts, w1, b1, w2, b2, epsv)


def _heads_body(h_r, m_r, pr_r, wmu_r, bmu_r, wlv_r, blv_r,
                wp1_r, bp1_r, wp2_r, bp2_r, loss_r, kl_r):
    hm = h_r[...].reshape(B, NPER, D)
    m = m_r[...]
    gsum = jnp.sum(hm * m[:, :, None], axis=1)
    gcnt = jnp.sum(m, axis=1)
    g = gsum / jnp.maximum(gcnt, 1.0)[:, None]
    mu = jnp.dot(g, wmu_r[...], preferred_element_type=jnp.float32) + bmu_r[...]
    lv = jnp.dot(g, wlv_r[...], preferred_element_type=jnp.float32) + blv_r[...]
    kl_r[0, 0] = -0.5 * jnp.mean(
        jnp.sum(1.0 + lv - mu * mu - jnp.exp(lv), axis=-1))
    act = jnp.maximum(
        jnp.dot(mu, wp1_r[...], preferred_element_type=jnp.float32) + bp1_r[...],
        0.0)
    pred = (jnp.dot(act, wp2_r[...], preferred_element_type=jnp.float32)
            + bp2_r[...])
    loss_r[0, 0] = jnp.mean((pred - pr_r[...]) ** 2)


def _heads(h, mask_f, props, w_mu, b_mu, w_lv, b_lv, w_p1, b_p1, w_p2, b_p2):
    return pl.pallas_call(
        _heads_body,
        out_specs=[pl.BlockSpec(memory_space=pltpu.SMEM),
                   pl.BlockSpec(memory_space=pltpu.SMEM)],
        out_shape=[jax.ShapeDtypeStruct((1, 1), jnp.float32),
                   jax.ShapeDtypeStruct((1, 1), jnp.float32)],
    )(h, mask_f, props, w_mu, b_mu, w_lv, b_lv, w_p1, b_p1, w_p2, b_p2)


def kernel(x, x_pieces, x_pos, edge_index, edge_attr, atom_mask, props,
           atom_emb, piece_emb, pos_emb, edge_emb, Ws1, bs1, Ws2, bs2, eps,
           W_mu, b_mu, W_lv, b_lv, W_p1, b_p1, W_p2, b_p2):
    src = edge_index[0].astype(jnp.int32)
    dste = edge_index[1].astype(jnp.int32)
    pad_e = _EPE - E
    packed = jnp.sort((edge_attr.astype(jnp.int32) * N + src) * 16384 + dste)
    sgidx = packed >> 14
    sdst = packed & 16383
    gidx_e = jnp.concatenate(
        [sgidx, jnp.full((pad_e,), 4 * N, jnp.int32)]).reshape(NW, _KE, CH)
    dst_e = jnp.concatenate(
        [sdst, jnp.full((pad_e,), N, jnp.int32)]).reshape(NW, _KE, CH)

    eemb3 = edge_emb.reshape(4, 1, D)
    h, hrel = _emb_build(
        x.reshape(N, 1), x_pieces.reshape(N, 1), x_pos.reshape(N, 1),
        atom_emb, piece_emb, pos_emb, eemb3)

    for t in range(T):
        parts = _make_sc_segsum(_KE)(gidx_e, dst_e, hrel)
        epsv = eps[t].reshape(1, 1)
        b1 = bs1[t].reshape(1, D)
        b2 = bs2[t].reshape(1, D)
        if t < T - 1:
            h, hrel = _mlp_table(h, parts, Ws1[t], b1, Ws2[t], b2, epsv, eemb3)
        else:
            h = _mlp_final(h, parts, Ws1[t], b1, Ws2[t], b2, epsv)

    loss, kl = _heads(
        h, atom_mask.astype(jnp.float32), props,
        W_mu, b_mu.reshape(1, LAT), W_lv, b_lv.reshape(1, LAT),
        W_p1, b_p1.reshape(1, D), W_p2, b_p2.reshape(1, NPROP))
    return (loss[0, 0], kl[0, 0])


# speedup vs baseline: 1.2364x; 1.2364x over previous
import functools

import jax
import jax.numpy as jnp
from jax import lax
from jax.experimental import pallas as pl
from jax.experimental.pallas import tpu as pltpu
from jax.experimental.pallas import tpu_sc as plsc

B = 100
NPER = 100
N = B * NPER
E = 160000
D = 128
LAT = 56
T = 4
ATOM_V = 100
PIECE_V = 500
POS_V = 100
NPROP = 3

NC = 2
NS = 16
NW = NC * NS
CH = 128
SPM_ROWS = 10240
PN = N
ZROWS = 32
NBUF = 2
OUT_PER_SUB = 640

_EPE = 163840
_KE = _EPE // (NW * CH)
HREL_ROWS = 4 * N + 8


@functools.lru_cache(maxsize=None)
def _make_sc_segsum(k, dcols=D, ch=CH):
    mesh = plsc.VectorSubcoreMesh(core_axis_name="c", subcore_axis_name="s",
                                  num_cores=NC, num_subcores=NS)

    @functools.partial(
        pl.kernel,
        out_type=jax.ShapeDtypeStruct((NC, PN, D), jnp.float32),
        mesh=mesh,
        scratch_types=[
            pltpu.VMEM((k, ch), jnp.int32),
            pltpu.VMEM((k, ch), jnp.int32),
            pltpu.VMEM((NBUF * ch, dcols), jnp.float32),
            pltpu.VMEM((ZROWS, D), jnp.float32),
            pltpu.VMEM_SHARED((SPM_ROWS, D), jnp.float32),
            pltpu.SemaphoreType.DMA,
            pltpu.SemaphoreType.DMA,
            pltpu.SemaphoreType.DMA,
            pltpu.SemaphoreType.DMA,
        ],
    )
    def sc_segsum(gidx_hbm, dst_hbm, table_hbm, out_hbm,
                  gidx_v, dst_v, ring, zbuf, acc, s0, s1, s2, s3):
        c = lax.axis_index("c")
        s = lax.axis_index("s")
        wid = c * NS + s
        sems = (s0, s1)
        ssems = (s2, s3)

        pltpu.sync_copy(gidx_hbm.at[wid], gidx_v)
        pltpu.sync_copy(dst_hbm.at[wid], dst_v)

        def gather(j, b):
            return pltpu.make_async_copy(
                table_hbm.at[gidx_v.at[j]],
                ring.at[pl.ds(b * ch, ch)], sems[b])

        for b in range(NBUF):
            gather(b, b).start()

        zv = jnp.zeros((16,), jnp.float32)

        @pl.loop(0, ZROWS)
        def _(r):
            for u in range(D // 16):
                zbuf[r, pl.ds(u * 16, 16)] = zv

        zper = SPM_ROWS // NS

        @pl.loop(0, zper // ZROWS)
        def _(j):
            pltpu.sync_copy(zbuf, acc.at[pl.ds(s * zper + j * ZROWS, ZROWS)])

        plsc.subcore_barrier()

        @pl.loop(0, k, step=NBUF)
        def _(j):
            for b in range(NBUF):
                gather(j + b, b).wait()
                pltpu.sync_copy(ring.at[pl.ds(b * ch, ch)],
                                acc.at[dst_v.at[j + b]], add=True)

                @pl.when(j + b + NBUF < k)
                def _():
                    gather(j + b + NBUF, b).start()

        plsc.subcore_barrier()

        @pl.when(s < NS - 1)
        def _():
            pltpu.sync_copy(
                acc.at[pl.ds(s * OUT_PER_SUB, OUT_PER_SUB)],
                out_hbm.at[c].at[pl.ds(s * OUT_PER_SUB, OUT_PER_SUB)])

        @pl.when(s == NS - 1)
        def _():
            pltpu.sync_copy(
                acc.at[pl.ds((NS - 1) * OUT_PER_SUB, N - (NS - 1) * OUT_PER_SUB)],
                out_hbm.at[c].at[pl.ds((NS - 1) * OUT_PER_SUB,
                                       N - (NS - 1) * OUT_PER_SUB)])

    return sc_segsum


_RB = 400
_NB = N // _RB


def _emb_body(xi_r, xp_r, xo_r, ae_r, pe_r, oe_r, ee_r, h_r, hr_r, hs):
    @pl.when(pl.program_id(1) == 0)
    def _():
        def oh_dot(idx_r, tbl_r, v):
            oh = (idx_r[...][:, 0][:, None] ==
                  lax.broadcasted_iota(jnp.int32, (_RB, v), 1)
                  ).astype(jnp.float32)
            return jnp.dot(oh, tbl_r[...], preferred_element_type=jnp.float32)

        h0 = (oh_dot(xi_r, ae_r, ATOM_V) + oh_dot(xp_r, pe_r, PIECE_V)
              + oh_dot(xo_r, oe_r, POS_V))
        hs[...] = h0
        h_r[...] = h0

    hr_r[...] = jnp.maximum(hs[...] + ee_r[...].reshape(1, D), 0.0)


def _emb_build(xi, xp, xo, atom_emb, piece_emb, pos_emb, eemb):
    row1 = lambda i, a: (i, 0)
    fix = lambda i, a: (0, 0)
    return pl.pallas_call(
        _emb_body,
        grid=(_NB, 4),
        in_specs=[
            pl.BlockSpec((_RB, 1), row1),
            pl.BlockSpec((_RB, 1), row1),
            pl.BlockSpec((_RB, 1), row1),
            pl.BlockSpec((ATOM_V, D), fix),
            pl.BlockSpec((PIECE_V, D), fix),
            pl.BlockSpec((POS_V, D), fix),
            pl.BlockSpec((1, 1, D), lambda i, a: (a, 0, 0)),
        ],
        out_specs=[
            pl.BlockSpec((_RB, D), row1),
            pl.BlockSpec((_RB, D), lambda i, a: (a * _NB + i, 0)),
        ],
        out_shape=[
            jax.ShapeDtypeStruct((N, D), jnp.float32),
            jax.ShapeDtypeStruct((HREL_ROWS, D), jnp.float32),
        ],
        scratch_shapes=[pltpu.VMEM((_RB, D), jnp.float32)],
    )(xi, xp, xo, atom_emb, piece_emb, pos_emb, eemb)


def _mlp_math(h_r, parts_r, w1_r, b1_r, w2_r, b2_r, e_r):
    pre = (1.0 + e_r[0, 0]) * h_r[...] + parts_r[0] + parts_r[1]
    act = jnp.maximum(
        jnp.dot(pre, w1_r[...], preferred_element_type=jnp.float32) + b1_r[...],
        0.0)
    return jnp.dot(act, w2_r[...], preferred_element_type=jnp.float32) + b2_r[...]


def _mlp_table_body(h_r, parts_r, w1_r, b1_r, w2_r, b2_r, e_r, ee_r,
                    hn_r, hr_r, hs):
    @pl.when(pl.program_id(1) == 0)
    def _():
        hn = _mlp_math(h_r, parts_r, w1_r, b1_r, w2_r, b2_r, e_r)
        hs[...] = hn
        hn_r[...] = hn

    hr_r[...] = jnp.maximum(hs[...] + ee_r[...].reshape(1, D), 0.0)


def _mlp_table(h, parts, w1, b1, w2, b2, epsv, eemb):
    row = lambda i, a: (i, 0)
    fix = lambda i, a: (0, 0)
    return pl.pallas_call(
        _mlp_table_body,
        grid=(_NB, 4),
        in_specs=[
            pl.BlockSpec((_RB, D), row),
            pl.BlockSpec((NC, _RB, D), lambda i, a: (0, i, 0)),
            pl.BlockSpec((D, D), fix),
            pl.BlockSpec((1, D), fix),
            pl.BlockSpec((D, D), fix),
            pl.BlockSpec((1, D), fix),
            pl.BlockSpec(memory_space=pltpu.SMEM),
            pl.BlockSpec((1, 1, D), lambda i, a: (a, 0, 0)),
        ],
        out_specs=[
            pl.BlockSpec((_RB, D), row),
            pl.BlockSpec((_RB, D), lambda i, a: (a * _NB + i, 0)),
        ],
        out_shape=[
            jax.ShapeDtypeStruct((N, D), jnp.float32),
            jax.ShapeDtypeStruct((HREL_ROWS, D), jnp.float32),
        ],
        scratch_shapes=[pltpu.VMEM((_RB, D), jnp.float32)],
    )(h, parts, w1, b1, w2, b2, epsv, eemb)


def _mlp_final_body(h_r, parts_r, w1_r, b1_r, w2_r, b2_r, e_r, hn_r):
    hn_r[...] = _mlp_math(h_r, parts_r, w1_r, b1_r, w2_r, b2_r, e_r)


def _mlp_final(h, parts, w1, b1, w2, b2, epsv):
    row = lambda i: (i, 0)
    fix = lambda i: (0, 0)
    return pl.pallas_call(
        _mlp_final_body,
        grid=(_NB,),
        in_specs=[
            pl.BlockSpec((_RB, D), row),
            pl.BlockSpec((NC, _RB, D), lambda i: (0, i, 0)),
            pl.BlockSpec((D, D), fix),
            pl.BlockSpec((1, D), fix),
            pl.BlockSpec((D, D), fix),
            pl.BlockSpec((1, D), fix),
            pl.BlockSpec(memory_space=pltpu.SMEM),
        ],
        out_specs=pl.BlockSpec((_RB, D), row),
        out_shape=jax.ShapeDtypeStruct((N, D), jnp.float32),
    )(h, parts, w1, b1, w2, b2, epsv)


def _heads_body(h_r, m_r, pr_r, wmu_r, bmu_r, wlv_r, blv_r,
                wp1_r, bp1_r, wp2_r, bp2_r, loss_r, kl_r):
    hm = h_r[...].reshape(B, NPER, D)
    m = m_r[...]
    gsum = jnp.sum(hm * m[:, :, None], axis=1)
    gcnt = jnp.sum(m, axis=1)
    g = gsum / jnp.maximum(gcnt, 1.0)[:, None]
    mu = jnp.dot(g, wmu_r[...], preferred_element_type=jnp.float32) + bmu_r[...]
    lv = jnp.dot(g, wlv_r[...], preferred_element_type=jnp.float32) + blv_r[...]
    kl_r[0, 0] = -0.5 * jnp.mean(
        jnp.sum(1.0 + lv - mu * mu - jnp.exp(lv), axis=-1))
    act = jnp.maximum(
        jnp.dot(mu, wp1_r[...], preferred_element_type=jnp.float32) + bp1_r[...],
        0.0)
    pred = (jnp.dot(act, wp2_r[...], preferred_element_type=jnp.float32)
            + bp2_r[...])
    loss_r[0, 0] = jnp.mean((pred - pr_r[...]) ** 2)


def _heads(h, mask_f, props, w_mu, b_mu, w_lv, b_lv, w_p1, b_p1, w_p2, b_p2):
    return pl.pallas_call(
        _heads_body,
        out_specs=[pl.BlockSpec(memory_space=pltpu.SMEM),
                   pl.BlockSpec(memory_space=pltpu.SMEM)],
        out_shape=[jax.ShapeDtypeStruct((1, 1), jnp.float32),
                   jax.ShapeDtypeStruct((1, 1), jnp.float32)],
    )(h, mask_f, props, w_mu, b_mu, w_lv, b_lv, w_p1, b_p1, w_p2, b_p2)


def kernel(x, x_pieces, x_pos, edge_index, edge_attr, atom_mask, props,
           atom_emb, piece_emb, pos_emb, edge_emb, Ws1, bs1, Ws2, bs2, eps,
           W_mu, b_mu, W_lv, b_lv, W_p1, b_p1, W_p2, b_p2):
    src = edge_index[0].astype(jnp.int32)
    dste = edge_index[1].astype(jnp.int32)
    pad_e = _EPE - E
    sgidx, sdst = lax.sort_key_val(edge_attr.astype(jnp.int32) * N + src, dste)
    gidx_e = jnp.concatenate(
        [sgidx, jnp.full((pad_e,), 4 * N, jnp.int32)]
    ).reshape(_KE, NW, CH).swapaxes(0, 1)
    dst_e = jnp.concatenate(
        [sdst, jnp.full((pad_e,), N, jnp.int32)]
    ).reshape(_KE, NW, CH).swapaxes(0, 1)

    eemb3 = edge_emb.reshape(4, 1, D)
    h, hrel = _emb_build(
        x.reshape(N, 1), x_pieces.reshape(N, 1), x_pos.reshape(N, 1),
        atom_emb, piece_emb, pos_emb, eemb3)

    for t in range(T):
        parts = _make_sc_segsum(_KE)(gidx_e, dst_e, hrel)
        epsv = eps[t].reshape(1, 1)
        b1 = bs1[t].reshape(1, D)
        b2 = bs2[t].reshape(1, D)
        if t < T - 1:
            h, hrel = _mlp_table(h, parts, Ws1[t], b1, Ws2[t], b2, epsv, eemb3)
        else:
            h = _mlp_final(h, parts, Ws1[t], b1, Ws2[t], b2, epsv)

    loss, kl = _heads(
        h, atom_mask.astype(jnp.float32), props,
        W_mu, b_mu.reshape(1, LAT), W_lv, b_lv.reshape(1, LAT),
        W_p1, b_p1.reshape(1, D), W_p2, b_p2.reshape(1, NPROP))
    return (loss[0, 0], kl[0, 0])


# speedup vs baseline: 1.2867x; 1.0407x over previous
import functools

import jax
import jax.numpy as jnp
from jax import lax
from jax.experimental import pallas as pl
from jax.experimental.pallas import tpu as pltpu
from jax.experimental.pallas import tpu_sc as plsc

B = 100
NPER = 100
N = B * NPER
E = 160000
D = 128
LAT = 56
T = 4
ATOM_V = 100
PIECE_V = 500
POS_V = 100
NPROP = 3

NC = 2
NS = 16
NW = NC * NS
CH = 128
SPM_ROWS = 10240
PN = N
ZROWS = 32
NBUF = 2
OUT_PER_SUB = 640

_EPE = 163840
_KE = _EPE // (NW * CH)
HREL_ROWS = 4 * N + 8


@functools.lru_cache(maxsize=None)
def _make_sc_segsum(k, dcols=D, ch=CH):
    mesh = plsc.VectorSubcoreMesh(core_axis_name="c", subcore_axis_name="s",
                                  num_cores=NC, num_subcores=NS)

    @functools.partial(
        pl.kernel,
        out_type=jax.ShapeDtypeStruct((NC, PN, D), jnp.float32),
        mesh=mesh,
        scratch_types=[
            pltpu.VMEM((k, ch), jnp.int32),
            pltpu.VMEM((k, ch), jnp.int32),
            pltpu.VMEM((NBUF * ch, dcols), jnp.float32),
            pltpu.VMEM((ZROWS, D), jnp.float32),
            pltpu.VMEM_SHARED((SPM_ROWS, D), jnp.float32),
            pltpu.SemaphoreType.DMA,
            pltpu.SemaphoreType.DMA,
            pltpu.SemaphoreType.DMA,
            pltpu.SemaphoreType.DMA,
        ],
    )
    def sc_segsum(gidx_hbm, dst_hbm, table_hbm, out_hbm,
                  gidx_v, dst_v, ring, zbuf, acc, s0, s1, s2, s3):
        c = lax.axis_index("c")
        s = lax.axis_index("s")
        wid = c * NS + s
        sems = (s0, s1)
        ssems = (s2, s3)

        pltpu.sync_copy(gidx_hbm.at[wid], gidx_v)
        pltpu.sync_copy(dst_hbm.at[wid], dst_v)

        def gather(j, b):
            return pltpu.make_async_copy(
                table_hbm.at[gidx_v.at[j]],
                ring.at[pl.ds(b * ch, ch)], sems[b])

        for b in range(NBUF):
            gather(b, b).start()

        zv = jnp.zeros((16,), jnp.float32)

        @pl.loop(0, ZROWS)
        def _(r):
            for u in range(D // 16):
                zbuf[r, pl.ds(u * 16, 16)] = zv

        zper = SPM_ROWS // NS

        @pl.loop(0, zper // ZROWS)
        def _(j):
            pltpu.sync_copy(zbuf, acc.at[pl.ds(s * zper + j * ZROWS, ZROWS)])

        plsc.subcore_barrier()

        @pl.loop(0, k, step=NBUF)
        def _(j):
            for b in range(NBUF):
                gather(j + b, b).wait()
                pltpu.sync_copy(ring.at[pl.ds(b * ch, ch)],
                                acc.at[dst_v.at[j + b]], add=True)

                @pl.when(j + b + NBUF < k)
                def _():
                    gather(j + b + NBUF, b).start()

        plsc.subcore_barrier()

        @pl.when(s < NS - 1)
        def _():
            pltpu.sync_copy(
                acc.at[pl.ds(s * OUT_PER_SUB, OUT_PER_SUB)],
                out_hbm.at[c].at[pl.ds(s * OUT_PER_SUB, OUT_PER_SUB)])

        @pl.when(s == NS - 1)
        def _():
            pltpu.sync_copy(
                acc.at[pl.ds((NS - 1) * OUT_PER_SUB, N - (NS - 1) * OUT_PER_SUB)],
                out_hbm.at[c].at[pl.ds((NS - 1) * OUT_PER_SUB,
                                       N - (NS - 1) * OUT_PER_SUB)])

    return sc_segsum


_RB = 400
_NB = N // _RB


def _emb_body(xi_r, xp_r, xo_r, ae_r, pe_r, oe_r, ee_r, h_r, hr_r, hs):
    @pl.when(pl.program_id(1) == 0)
    def _():
        def oh_dot(idx_r, tbl_r, v):
            oh = (idx_r[...][:, 0][:, None] ==
                  lax.broadcasted_iota(jnp.int32, (_RB, v), 1)
                  ).astype(jnp.float32)
            return jnp.dot(oh, tbl_r[...], preferred_element_type=jnp.float32)

        h0 = (oh_dot(xi_r, ae_r, ATOM_V) + oh_dot(xp_r, pe_r, PIECE_V)
              + oh_dot(xo_r, oe_r, POS_V))
        hs[...] = h0
        h_r[...] = h0

    hr_r[...] = jnp.maximum(hs[...] + ee_r[...].reshape(1, D), 0.0)


def _emb_build(xi, xp, xo, atom_emb, piece_emb, pos_emb, eemb):
    row1 = lambda i, a: (i, 0)
    fix = lambda i, a: (0, 0)
    return pl.pallas_call(
        _emb_body,
        grid=(_NB, 4),
        in_specs=[
            pl.BlockSpec((_RB, 1), row1),
            pl.BlockSpec((_RB, 1), row1),
            pl.BlockSpec((_RB, 1), row1),
            pl.BlockSpec((ATOM_V, D), fix),
            pl.BlockSpec((PIECE_V, D), fix),
            pl.BlockSpec((POS_V, D), fix),
            pl.BlockSpec((1, 1, D), lambda i, a: (a, 0, 0)),
        ],
        out_specs=[
            pl.BlockSpec((_RB, D), row1),
            pl.BlockSpec((_RB, D), lambda i, a: (a * _NB + i, 0)),
        ],
        out_shape=[
            jax.ShapeDtypeStruct((N, D), jnp.float32),
            jax.ShapeDtypeStruct((HREL_ROWS, D), jnp.float32),
        ],
        scratch_shapes=[pltpu.VMEM((_RB, D), jnp.float32)],
    )(xi, xp, xo, atom_emb, piece_emb, pos_emb, eemb)


def _mlp_math(h_r, parts_r, w1_r, b1_r, w2_r, b2_r, e_r):
    pre = (1.0 + e_r[0, 0]) * h_r[...] + parts_r[0] + parts_r[1]
    act = jnp.maximum(
        jnp.dot(pre, w1_r[...], preferred_element_type=jnp.float32) + b1_r[...],
        0.0)
    return jnp.dot(act, w2_r[...], preferred_element_type=jnp.float32) + b2_r[...]


def _mlp_table_body(h_r, parts_r, w1_r, b1_r, w2_r, b2_r, e_r, ee_r,
                    hn_r, hr_r, hs):
    @pl.when(pl.program_id(1) == 0)
    def _():
        hn = _mlp_math(h_r, parts_r, w1_r, b1_r, w2_r, b2_r, e_r)
        hs[...] = hn
        hn_r[...] = hn

    hr_r[...] = jnp.maximum(hs[...] + ee_r[...].reshape(1, D), 0.0)


def _mlp_table(h, parts, w1, b1, w2, b2, epsv, eemb):
    row = lambda i, a: (i, 0)
    fix = lambda i, a: (0, 0)
    return pl.pallas_call(
        _mlp_table_body,
        grid=(_NB, 4),
        in_specs=[
            pl.BlockSpec((_RB, D), row),
            pl.BlockSpec((NC, _RB, D), lambda i, a: (0, i, 0)),
            pl.BlockSpec((D, D), fix),
            pl.BlockSpec((1, D), fix),
            pl.BlockSpec((D, D), fix),
            pl.BlockSpec((1, D), fix),
            pl.BlockSpec(memory_space=pltpu.SMEM),
            pl.BlockSpec((1, 1, D), lambda i, a: (a, 0, 0)),
        ],
        out_specs=[
            pl.BlockSpec((_RB, D), row),
            pl.BlockSpec((_RB, D), lambda i, a: (a * _NB + i, 0)),
        ],
        out_shape=[
            jax.ShapeDtypeStruct((N, D), jnp.float32),
            jax.ShapeDtypeStruct((HREL_ROWS, D), jnp.float32),
        ],
        scratch_shapes=[pltpu.VMEM((_RB, D), jnp.float32)],
    )(h, parts, w1, b1, w2, b2, epsv, eemb)


def _mlp_final_body(h_r, parts_r, w1_r, b1_r, w2_r, b2_r, e_r, hn_r):
    hn_r[...] = _mlp_math(h_r, parts_r, w1_r, b1_r, w2_r, b2_r, e_r)


def _mlp_final(h, parts, w1, b1, w2, b2, epsv):
    row = lambda i: (i, 0)
    fix = lambda i: (0, 0)
    return pl.pallas_call(
        _mlp_final_body,
        grid=(_NB,),
        in_specs=[
            pl.BlockSpec((_RB, D), row),
            pl.BlockSpec((NC, _RB, D), lambda i: (0, i, 0)),
            pl.BlockSpec((D, D), fix),
            pl.BlockSpec((1, D), fix),
            pl.BlockSpec((D, D), fix),
            pl.BlockSpec((1, D), fix),
            pl.BlockSpec(memory_space=pltpu.SMEM),
        ],
        out_specs=pl.BlockSpec((_RB, D), row),
        out_shape=jax.ShapeDtypeStruct((N, D), jnp.float32),
    )(h, parts, w1, b1, w2, b2, epsv)


def _heads_body(h_r, m_r, pr_r, wmu_r, bmu_r, wlv_r, blv_r,
                wp1_r, bp1_r, wp2_r, bp2_r, loss_r, kl_r):
    hm = h_r[...].reshape(B, NPER, D)
    m = m_r[...]
    gsum = jnp.sum(hm * m[:, :, None], axis=1)
    gcnt = jnp.sum(m, axis=1)
    g = gsum / jnp.maximum(gcnt, 1.0)[:, None]
    mu = jnp.dot(g, wmu_r[...], preferred_element_type=jnp.float32) + bmu_r[...]
    lv = jnp.dot(g, wlv_r[...], preferred_element_type=jnp.float32) + blv_r[...]
    kl_r[0, 0] = -0.5 * jnp.mean(
        jnp.sum(1.0 + lv - mu * mu - jnp.exp(lv), axis=-1))
    act = jnp.maximum(
        jnp.dot(mu, wp1_r[...], preferred_element_type=jnp.float32) + bp1_r[...],
        0.0)
    pred = (jnp.dot(act, wp2_r[...], preferred_element_type=jnp.float32)
            + bp2_r[...])
    loss_r[0, 0] = jnp.mean((pred - pr_r[...]) ** 2)


def _heads(h, mask_f, props, w_mu, b_mu, w_lv, b_lv, w_p1, b_p1, w_p2, b_p2):
    return pl.pallas_call(
        _heads_body,
        out_specs=[pl.BlockSpec(memory_space=pltpu.SMEM),
                   pl.BlockSpec(memory_space=pltpu.SMEM)],
        out_shape=[jax.ShapeDtypeStruct((1, 1), jnp.float32),
                   jax.ShapeDtypeStruct((1, 1), jnp.float32)],
    )(h, mask_f, props, w_mu, b_mu, w_lv, b_lv, w_p1, b_p1, w_p2, b_p2)


def kernel(x, x_pieces, x_pos, edge_index, edge_attr, atom_mask, props,
           atom_emb, piece_emb, pos_emb, edge_emb, Ws1, bs1, Ws2, bs2, eps,
           W_mu, b_mu, W_lv, b_lv, W_p1, b_p1, W_p2, b_p2):
    src = edge_index[0].astype(jnp.int32)
    dste = edge_index[1].astype(jnp.int32)
    pad_e = _EPE - E
    sgidx, sdst = lax.sort_key_val(edge_attr.astype(jnp.int32) * N + src, dste)
    gidx_e = jnp.concatenate(
        [sgidx, jnp.full((pad_e,), 4 * N, jnp.int32)]).reshape(NW, _KE, CH)
    dst_e = jnp.concatenate(
        [sdst, jnp.full((pad_e,), N, jnp.int32)]).reshape(NW, _KE, CH)

    eemb3 = edge_emb.reshape(4, 1, D)
    h, hrel = _emb_build(
        x.reshape(N, 1), x_pieces.reshape(N, 1), x_pos.reshape(N, 1),
        atom_emb, piece_emb, pos_emb, eemb3)

    for t in range(T):
        parts = _make_sc_segsum(_KE)(gidx_e, dst_e, hrel)
        epsv = eps[t].reshape(1, 1)
        b1 = bs1[t].reshape(1, D)
        b2 = bs2[t].reshape(1, D)
        if t < T - 1:
            h, hrel = _mlp_table(h, parts, Ws1[t], b1, Ws2[t], b2, epsv, eemb3)
        else:
            h = _mlp_final(h, parts, Ws1[t], b1, Ws2[t], b2, epsv)

    loss, kl = _heads(
        h, atom_mask.astype(jnp.float32), props,
        W_mu, b_mu.reshape(1, LAT), W_lv, b_lv.reshape(1, LAT),
        W_p1, b_p1.reshape(1, D), W_p2, b_p2.reshape(1, NPROP))
    return (loss[0, 0], kl[0, 0])
